# baseline (speedup 1.0000x reference)
import jax
import jax.numpy as jnp
from jax import lax
from jax.experimental import pallas as pl
from jax.experimental.pallas import tpu as pltpu
from jax.experimental.pallas import tpu_sc as plsc

VOCAB = 49408
NUM_POS = 77
EMBED_DIM = 768
BATCH = 1024
ROWS = BATCH * NUM_POS

_INFO = plsc.get_sparse_core_info()
_NC = _INFO.num_cores
_NS = _INFO.num_subcores
_NW = _NC * _NS
_RPW = ROWS // _NW
_IDS = 616
_NSTAGE = _RPW // _IDS


def _gather_body(tok_hbm, table_hbm, out_hbm, idx_v, sem_i, sem_d):
    wid = lax.axis_index("s") * _NC + lax.axis_index("c")
    r0 = wid * _RPW
    b0 = wid * (BATCH // _NW)

    pltpu.async_copy(tok_hbm.at[pl.ds(r0, _RPW)], idx_v, sem_i).wait()

    def group(g, carry):
        br, pp = carry
        vec = idx_v[pl.ds(g * 16, 16)]
        for j in range(16):
            t = vec[j]
            pltpu.async_copy(table_hbm.at[t], out_hbm.at[br, pp], sem_d)
            pp = pp + 1
            wrap = pp == NUM_POS
            br = lax.select(wrap, br + 1, br)
            pp = lax.select(wrap, 0, pp)
        return (br, pp)

    lax.fori_loop(0, _RPW // 16, group, (b0, 0))

    def drain(i, carry):
        pltpu.make_async_copy(
            table_hbm.at[0], out_hbm.at[b0, 0], sem_d
        ).wait()
        return carry

    lax.fori_loop(0, _RPW, drain, 0)


def _add_body(g_ref, pos_ref, out_ref):
    out_ref[...] = g_ref[...] + pos_ref[...]


@jax.jit
def kernel(input_tokens, token_table, pos_table):
    mesh = plsc.VectorSubcoreMesh(core_axis_name="c", subcore_axis_name="s")
    gathered = pl.kernel(
        _gather_body,
        mesh=mesh,
        out_type=jax.ShapeDtypeStruct((BATCH, NUM_POS, EMBED_DIM), jnp.float32),
        scratch_types=[
            pltpu.VMEM((_RPW,), jnp.int32),
            pltpu.SemaphoreType.DMA,
            pltpu.SemaphoreType.DMA,
        ],
    )(input_tokens.astype(jnp.int32).reshape(ROWS), token_table)

    grid = (BATCH // 8,)
    return pl.pallas_call(
        _add_body,
        grid=grid,
        in_specs=[
            pl.BlockSpec((8, NUM_POS, EMBED_DIM), lambda i: (i, 0, 0)),
            pl.BlockSpec((1, NUM_POS, EMBED_DIM), lambda i: (0, 0, 0)),
        ],
        out_specs=pl.BlockSpec((8, NUM_POS, EMBED_DIM), lambda i: (i, 0, 0)),
        out_shape=jax.ShapeDtypeStruct((BATCH, NUM_POS, EMBED_DIM), jnp.float32),
    )(gathered, pos_table[None])


# speedup vs baseline: 11.2647x; 11.2647x over previous
import jax
import jax.numpy as jnp
from jax import lax
from jax.experimental import pallas as pl
from jax.experimental.pallas import tpu as pltpu
from jax.experimental.pallas import tpu_sc as plsc

VOCAB = 49408
NUM_POS = 77
POS_PAD = 80
EMBED_DIM = 768
BATCH = 1024

_INFO = plsc.get_sparse_core_info()
_NC = _INFO.num_cores
_NS = _INFO.num_subcores
_NW = _NC * _NS
_BPW = BATCH // _NW


def _gather_body(tok_hbm, table_hbm, out_hbm, idx_v, spmem, sem_i, sem_d, sem_o):
    c = lax.axis_index("c")
    s = lax.axis_index("s")
    wid = s * _NC + c
    b0 = wid * _BPW

    pltpu.async_copy(
        tok_hbm.at[pl.ds(wid * _BPW * POS_PAD, _BPW * POS_PAD)], idx_v, sem_i
    ).wait()

    slot0 = s * (2 * POS_PAD)

    def batch(j, carry):
        base = slot0 + lax.rem(j, 2) * POS_PAD

        @pl.when(j >= 2)
        def _():
            pltpu.make_async_copy(
                spmem.at[pl.ds(slot0, POS_PAD), :],
                out_hbm.at[b0],
                sem_o,
            ).wait()

        for k in range(POS_PAD // 16):
            vec = idx_v[pl.ds(j * POS_PAD + k * 16, 16)]
            for l in range(16):
                p = k * 16 + l
                pltpu.async_copy(
                    table_hbm.at[pl.ds(vec[l], 1), :],
                    spmem.at[pl.ds(base + p, 1), :],
                    sem_d,
                )

        def drain_row(i, c2):
            pltpu.make_async_copy(
                table_hbm.at[pl.ds(0, 1), :],
                spmem.at[pl.ds(slot0, 1), :],
                sem_d,
            ).wait()
            return c2

        lax.fori_loop(0, POS_PAD, drain_row, 0)

        pltpu.async_copy(
            spmem.at[pl.ds(base, POS_PAD), :], out_hbm.at[b0 + j], sem_o
        )
        return carry

    lax.fori_loop(0, _BPW, batch, 0)

    def drain_out(i, c2):
        pltpu.make_async_copy(
            spmem.at[pl.ds(slot0, POS_PAD), :], out_hbm.at[b0], sem_o
        ).wait()
        return c2

    lax.fori_loop(0, 2, drain_out, 0)


def _add_body(g_ref, pos_ref, out_ref):
    out_ref[...] = g_ref[:, :NUM_POS, :] + pos_ref[...]


@jax.jit
def kernel(input_tokens, token_table, pos_table):
    tok = jnp.pad(
        input_tokens.astype(jnp.int32), ((0, 0), (0, POS_PAD - NUM_POS))
    ).reshape(BATCH * POS_PAD)

    mesh = plsc.VectorSubcoreMesh(core_axis_name="c", subcore_axis_name="s")
    gathered = pl.kernel(
        _gather_body,
        mesh=mesh,
        out_type=jax.ShapeDtypeStruct((BATCH, POS_PAD, EMBED_DIM), jnp.float32),
        scratch_types=[
            pltpu.VMEM((_BPW * POS_PAD,), jnp.int32),
            pltpu.VMEM_SHARED((_NS * 2 * POS_PAD, EMBED_DIM), jnp.float32),
            pltpu.SemaphoreType.DMA,
            pltpu.SemaphoreType.DMA,
            pltpu.SemaphoreType.DMA,
        ],
    )(tok, token_table)

    grid = (BATCH // 8,)
    return pl.pallas_call(
        _add_body,
        grid=grid,
        in_specs=[
            pl.BlockSpec((8, POS_PAD, EMBED_DIM), lambda i: (i, 0, 0)),
            pl.BlockSpec((1, NUM_POS, EMBED_DIM), lambda i: (0, 0, 0)),
        ],
        out_specs=pl.BlockSpec((8, NUM_POS, EMBED_DIM), lambda i: (i, 0, 0)),
        out_shape=jax.ShapeDtypeStruct((BATCH, NUM_POS, EMBED_DIM), jnp.float32),
    )(gathered, pos_table[None])


# speedup vs baseline: 11.8971x; 1.0561x over previous
import jax
import jax.numpy as jnp
from jax import lax
from jax.experimental import pallas as pl
from jax.experimental.pallas import tpu as pltpu
from jax.experimental.pallas import tpu_sc as plsc

VOCAB = 49408
NUM_POS = 77
POS_PAD = 80
EMBED_DIM = 768
BATCH = 1024

_INFO = plsc.get_sparse_core_info()
_NC = _INFO.num_cores
_NS = _INFO.num_subcores
_NW = _NC * _NS
_BPW = BATCH // _NW


def _gather_body(
    tok_hbm, table_hbm, out_hbm, idx_v, spmem, sem_i, sem_o,
    sem_d0, sem_d1, sem_d2, sem_d3,
):
    sem_d = (sem_d0, sem_d1, sem_d2, sem_d3)
    c = lax.axis_index("c")
    s = lax.axis_index("s")
    wid = s * _NC + c
    b0 = wid * _BPW

    pltpu.async_copy(
        tok_hbm.at[pl.ds(wid * _BPW * POS_PAD, _BPW * POS_PAD)], idx_v, sem_i
    ).wait()

    slot0 = s * (2 * POS_PAD)

    def batch(j, carry):
        base = slot0 + lax.rem(j, 2) * POS_PAD

        @pl.when(j >= 2)
        def _():
            pltpu.make_async_copy(
                spmem.at[pl.ds(slot0, POS_PAD), :],
                out_hbm.at[b0],
                sem_o,
            ).wait()

        for k in range(POS_PAD // 16):
            vec = idx_v[pl.ds(j * POS_PAD + k * 16, 16)]
            for l in range(16):
                p = k * 16 + l
                pltpu.async_copy(
                    table_hbm.at[pl.ds(vec[l], 1), :],
                    spmem.at[pl.ds(base + p, 1), :],
                    sem_d[p % 4],
                )

        def drain_row(i, c2):
            for q in range(4):
                pltpu.make_async_copy(
                    table_hbm.at[pl.ds(0, 1), :],
                    spmem.at[pl.ds(slot0, 1), :],
                    sem_d[q],
                ).wait()
            return c2

        lax.fori_loop(0, POS_PAD // 4, drain_row, 0)

        pltpu.async_copy(
            spmem.at[pl.ds(base, POS_PAD), :], out_hbm.at[b0 + j], sem_o
        )
        return carry

    lax.fori_loop(0, _BPW, batch, 0)

    def drain_out(i, c2):
        pltpu.make_async_copy(
            spmem.at[pl.ds(slot0, POS_PAD), :], out_hbm.at[b0], sem_o
        ).wait()
        return c2

    lax.fori_loop(0, 2, drain_out, 0)


def _add_body(g_ref, pos_ref, out_ref):
    out_ref[...] = g_ref[:, :NUM_POS, :] + pos_ref[...]


@jax.jit
def kernel(input_tokens, token_table, pos_table):
    tok = jnp.pad(
        input_tokens.astype(jnp.int32), ((0, 0), (0, POS_PAD - NUM_POS))
    ).reshape(BATCH * POS_PAD)

    mesh = plsc.VectorSubcoreMesh(core_axis_name="c", subcore_axis_name="s")
    gathered = pl.kernel(
        _gather_body,
        mesh=mesh,
        out_type=jax.ShapeDtypeStruct((BATCH, POS_PAD, EMBED_DIM), jnp.float32),
        scratch_types=[
            pltpu.VMEM((_BPW * POS_PAD,), jnp.int32),
            pltpu.VMEM_SHARED((_NS * 2 * POS_PAD, EMBED_DIM), jnp.float32),
            pltpu.SemaphoreType.DMA,
            pltpu.SemaphoreType.DMA,
            pltpu.SemaphoreType.DMA,
            pltpu.SemaphoreType.DMA,
            pltpu.SemaphoreType.DMA,
            pltpu.SemaphoreType.DMA,
        ],
    )(tok, token_table)

    grid = (BATCH // 8,)
    return pl.pallas_call(
        _add_body,
        grid=grid,
        in_specs=[
            pl.BlockSpec((8, POS_PAD, EMBED_DIM), lambda i: (i, 0, 0)),
            pl.BlockSpec((1, NUM_POS, EMBED_DIM), lambda i: (0, 0, 0)),
        ],
        out_specs=pl.BlockSpec((8, NUM_POS, EMBED_DIM), lambda i: (i, 0, 0)),
        out_shape=jax.ShapeDtypeStruct((BATCH, NUM_POS, EMBED_DIM), jnp.float32),
    )(gathered, pos_table[None])
